# baseline (speedup 1.0000x reference)
import numpy as np
import jax
import jax.numpy as jnp
from jax.experimental import pallas as pl
from jax.experimental.pallas import tpu as pltpu

_N = 10000
_HID = 128
_NF = 32
_CUT = 2.5
_NG = 20
_NGP = 32
_NI = 3
_BOX = 27.3

_T = 128
_S = 128
_NPAD = 10240
_NT = _NPAD // _T
_NC = _NPAD // _S

_off_np = np.linspace(0.0, _CUT, _NG).astype(np.float32)
_OFFS = np.full((_NGP,), 1e3, np.float32)
_OFFS[:_NG] = _off_np
_COEFF = np.float32(-0.5 / (_off_np[1] - _off_np[0]) ** 2)
_LOG2 = np.float32(np.log(2.0))
_PI = np.float32(np.pi)
_F32 = jnp.float32


def _sp(x):
    return jnp.logaddexp(x, 0.0) - _LOG2


def _emb_body(zf_ref, emb_ref, o_ref):
    zf = zf_ref[...]
    cls = jax.lax.broadcasted_iota(jnp.int32, (_T, 128), 1).astype(_F32)
    oh = (zf == cls).astype(_F32)
    o_ref[...] = jnp.dot(oh, emb_ref[...], preferred_element_type=_F32)


def _xl_body(h_ref, w_ref, o_ref):
    o_ref[...] = jnp.dot(h_ref[...], w_ref[...], preferred_element_type=_F32)


def _msg_body(cnt_ref, lst_ref, posr_ref, posc_ref, xl_ref, h_ref, offs_ref,
              w1_ref, b1_ref, w2_ref, b2_ref, l2w_ref, l2b_ref,
              lw_ref, lb_ref, o_ref):
    t = pl.program_id(0)
    pd = posr_ref[...]
    xd = pd[:, 0:1]
    yd = pd[:, 1:2]
    zd = pd[:, 2:3]
    sqd = xd * xd + yd * yd + zd * zd
    rowid = jax.lax.broadcasted_iota(jnp.int32, (_T, 1), 0) + t * _T
    w1 = w1_ref[...]
    b1 = b1_ref[0:1, :]
    w2 = w2_ref[...]
    b2 = b2_ref[0:1, :]
    offs = offs_ref[0:1, :].reshape(1, 1, _NGP)
    cnt = cnt_ref[t]

    def step(s, acc):
        c = lst_ref[t, s]
        base = c * _S
        ps = posc_ref[:, pl.ds(base, _S)]
        xs = ps[0:1, :]
        ys = ps[1:2, :]
        zs = ps[2:3, :]
        sqs = xs * xs + ys * ys + zs * zs
        dt = xd * xs + yd * ys + zd * zs
        d2 = sqd + sqs - 2.0 * dt
        colid = jax.lax.broadcasted_iota(jnp.int32, (1, _S), 1) + base
        m = (d2 < _CUT * _CUT) & (rowid != colid)
        dx = xs - xd
        dy = ys - yd
        dz = zs - zd
        ew = jnp.sqrt(dx * dx + dy * dy + dz * dz + 1e-12)
        dlt = ew[:, :, None] - offs
        ea = jnp.exp(_COEFF * (dlt * dlt)).reshape(_T * _S, _NGP)
        t1 = jnp.dot(ea, w1, preferred_element_type=_F32) + b1
        g = _sp(t1)
        wf = jnp.dot(g, w2, preferred_element_type=_F32) + b2
        cw = 0.5 * (jnp.cos((ew * _PI) / _CUT) + 1.0)
        fac = jnp.where(m, cw, 0.0)
        wf = (wf.reshape(_T, _S, _NF)) * fac[:, :, None]
        xls = xl_ref[pl.ds(base, _S), :]
        msg = xls[None, :, :] * wf
        return acc + jnp.sum(msg, axis=1)

    acc = jax.lax.fori_loop(0, cnt, step, jnp.zeros((_T, _NF), _F32))
    xc = jnp.dot(acc, l2w_ref[...], preferred_element_type=_F32) + l2b_ref[0:1, :]
    xc = _sp(xc)
    xc = jnp.dot(xc, lw_ref[...], preferred_element_type=_F32) + lb_ref[0:1, :]
    o_ref[...] = h_ref[...] + xc


def _head_body(h_ref, o1w_ref, o1b_ref, o2w_ref, o2b_ref, o_ref):
    t = pl.program_id(0)

    @pl.when(t == 0)
    def _():
        o_ref[...] = jnp.zeros_like(o_ref)

    h = h_ref[...]
    u = _sp(jnp.dot(h, o1w_ref[...], preferred_element_type=_F32) + o1b_ref[0:1, :])
    y = jnp.dot(u, o2w_ref[...], preferred_element_type=_F32) + o2b_ref[0:1, :]
    rowid = jax.lax.broadcasted_iota(jnp.int32, (_T, 1), 0) + t * _T
    y = jnp.where(rowid < _N, y, 0.0)
    o_ref[0:1, :] += jnp.sum(y, axis=0, keepdims=True)

    @pl.when(t == _NT - 1)
    def _():
        o_ref[...] = jnp.maximum(o_ref[...], 0.0)


def _pad8(b):
    return jnp.zeros((8, b.shape[0]), _F32).at[0].set(b.astype(_F32))


def _full(shape):
    return pl.BlockSpec(shape, lambda i, *_: tuple(0 for _ in shape))


def _rows(shape):
    return pl.BlockSpec(shape, lambda i, *_: (i,) + tuple(0 for _ in shape[1:]))


def kernel(z, pos, emb, mlp_w1, mlp_b1, mlp_w2, mlp_b2, lin1_w, lin2_w, lin2_b,
           lin_w, lin_b, out1_w, out1_b, out2_w, out2_b):
    pos = pos.astype(_F32)

    cell = jnp.clip((pos * (10.0 / _BOX)).astype(jnp.int32), 0, 9)

    def spread(v):
        return (v & 1) | ((v & 2) << 2) | ((v & 4) << 4) | ((v & 8) << 6)

    key = spread(cell[:, 0]) | (spread(cell[:, 1]) << 1) | (spread(cell[:, 2]) << 2)
    perm = jnp.argsort(key)
    pos_p = pos[perm]
    z_p = z[perm].astype(jnp.int32)

    posf = jnp.concatenate([pos_p, jnp.full((_NPAD - _N, 3), 1e6, _F32)], axis=0)
    zf = jnp.concatenate([z_p, jnp.zeros((_NPAD - _N,), jnp.int32)]) \
        .astype(_F32).reshape(_NPAD, 1)
    posr = jnp.pad(posf, ((0, 0), (0, 5)))
    posc = posr.T + 0.0

    pr = posf.reshape(_NT, _T, 3)
    vid = (jnp.arange(_NPAD).reshape(_NT, _T, 1)) < _N
    lo = jnp.min(jnp.where(vid, pr, jnp.inf), axis=1)
    hi = jnp.max(jnp.where(vid, pr, -jnp.inf), axis=1)
    gap = jnp.maximum(0.0, jnp.maximum(lo[:, None, :] - hi[None, :, :],
                                       lo[None, :, :] - hi[:, None, :]))
    act = jnp.sum(gap * gap, axis=-1) <= (_CUT + 1e-2) ** 2
    cnt = jnp.sum(act, axis=1).astype(jnp.int32)
    lst = jnp.argsort(~act, axis=1, stable=True).astype(jnp.int32)

    embp = jnp.pad(emb.astype(_F32), ((0, 128 - emb.shape[0]), (0, 0)))
    offsp = _pad8(jnp.asarray(_OFFS))
    h = pl.pallas_call(
        _emb_body,
        grid=(_NT,),
        in_specs=[_rows((_T, 1)), _full((128, _HID))],
        out_specs=_rows((_T, _HID)),
        out_shape=jax.ShapeDtypeStruct((_NPAD, _HID), _F32),
    )(zf, embp)

    for i in range(_NI):
        xl = pl.pallas_call(
            _xl_body,
            grid=(_NT,),
            in_specs=[_rows((_T, _HID)), _full((_HID, _NF))],
            out_specs=_rows((_T, _NF)),
            out_shape=jax.ShapeDtypeStruct((_NPAD, _NF), _F32),
        )(h, lin1_w[i].astype(_F32))

        w1p = jnp.pad(mlp_w1[i].astype(_F32), ((0, _NGP - _NG), (0, 0)))
        grid_spec = pltpu.PrefetchScalarGridSpec(
            num_scalar_prefetch=2,
            grid=(_NT,),
            in_specs=[
                _rows((_T, 8)),
                _full((8, _NPAD)),
                _full((_NPAD, _NF)),
                _rows((_T, _HID)),
                _full((8, _NGP)),
                _full((_NGP, _NF)),
                _full((8, _NF)),
                _full((_NF, _NF)),
                _full((8, _NF)),
                _full((_NF, _HID)),
                _full((8, _HID)),
                _full((_HID, _HID)),
                _full((8, _HID)),
            ],
            out_specs=_rows((_T, _HID)),
        )
        h = pl.pallas_call(
            _msg_body,
            grid_spec=grid_spec,
            out_shape=jax.ShapeDtypeStruct((_NPAD, _HID), _F32),
        )(cnt, lst, posr, posc, xl, h, offsp,
          w1p, _pad8(mlp_b1[i]), mlp_w2[i].astype(_F32), _pad8(mlp_b2[i]),
          lin2_w[i].astype(_F32), _pad8(lin2_b[i]),
          lin_w[i].astype(_F32), _pad8(lin_b[i]))

    o8 = pl.pallas_call(
        _head_body,
        grid=(_NT,),
        in_specs=[_rows((_T, _HID)), _full((_HID, _HID // 2)),
                  _full((8, _HID // 2)), _full((_HID // 2, _NF)),
                  _full((8, _NF))],
        out_specs=_full((8, _NF)),
        out_shape=jax.ShapeDtypeStruct((8, _NF), _F32),
    )(h, out1_w.astype(_F32), _pad8(out1_b), out2_w.astype(_F32), _pad8(out2_b))
    return o8[0:1]


# speedup vs baseline: 1.4562x; 1.4562x over previous
import numpy as np
import jax
import jax.numpy as jnp
from jax.experimental import pallas as pl
from jax.experimental.pallas import tpu as pltpu

_N = 10000
_HID = 128
_NF = 32
_CUT = 2.5
_NG = 20
_NGP = 32
_NI = 3
_BOX = 27.3

_T = 128
_S = 128
_S4 = _S // 4
_NPAD = 10240
_NT = _NPAD // _T
_NC = _NPAD // _S
_REAL = 125

_off_np = np.linspace(0.0, _CUT, _NG).astype(np.float32)
_OFFS = np.full((_NGP,), 1e3, np.float32)
_OFFS[:_NG] = _off_np
_COEFF = np.float32(-0.5 / (_off_np[1] - _off_np[0]) ** 2)
_LOG2 = np.float32(np.log(2.0))
_PI = np.float32(np.pi)
_F32 = jnp.float32


def _sp(x):
    return jnp.logaddexp(x, 0.0) - _LOG2


def _emb_body(zf_ref, emb_ref, o_ref):
    zf = zf_ref[...]
    cls = jax.lax.broadcasted_iota(jnp.int32, (_T, 128), 1).astype(_F32)
    oh = (zf == cls).astype(_F32)
    o_ref[...] = jnp.dot(oh, emb_ref[...], preferred_element_type=_F32)


def _xl_body(h_ref, w_ref, o_ref):
    o_ref[...] = jnp.dot(h_ref[...], w_ref[...], preferred_element_type=_F32)


def _msg_body(cnt_ref, lst_ref, posr_ref, xsp_ref, ysp_ref, zsp_ref, xlp_ref,
              h_ref, offs_ref, w1_ref, b1_ref, w2_ref, b2_ref, q_ref,
              l2w_ref, l2b_ref, lw_ref, lb_ref, o_ref):
    t = pl.program_id(0)
    pd = posr_ref[...]
    xd = pd[:, 0:1].reshape(_T, 1, 1)
    yd = pd[:, 1:2].reshape(_T, 1, 1)
    zd = pd[:, 2:3].reshape(_T, 1, 1)
    sqd = xd * xd + yd * yd + zd * zd
    rowid = jax.lax.broadcasted_iota(jnp.int32, (_T, 1, 1), 0) + t * _T
    lane_a = jax.lax.broadcasted_iota(jnp.int32, (1, _S4, 128), 2) // 32
    s4_i = jax.lax.broadcasted_iota(jnp.int32, (1, _S4, 128), 1)
    w1 = w1_ref[...]
    b1 = b1_ref[0:1, :].reshape(1, 1, 128)
    w2 = w2_ref[...]
    b2 = b2_ref[0:1, :].reshape(1, 1, 128)
    offs = offs_ref[0:1, :].reshape(1, 1, 128)
    cnt = cnt_ref[t]

    def step(s, acc):
        c = lst_ref[t, s]
        base = c * _S4
        xs = xsp_ref[pl.ds(base, _S4), :].reshape(1, _S4, 128)
        ys = ysp_ref[pl.ds(base, _S4), :].reshape(1, _S4, 128)
        zs = zsp_ref[pl.ds(base, _S4), :].reshape(1, _S4, 128)
        sqs = xs * xs + ys * ys + zs * zs
        dt = xd * xs + yd * ys + zd * zs
        d2 = sqd + sqs - 2.0 * dt
        colid = c * _S + s4_i * 4 + lane_a
        m = (d2 < _CUT * _CUT) & (rowid != colid)
        dx = xs - xd
        dy = ys - yd
        dz = zs - zd
        ew = jnp.sqrt(dx * dx + dy * dy + dz * dz + 1e-12)
        dlt = ew - offs
        ea = jnp.exp(_COEFF * (dlt * dlt)).reshape(_T * _S4, 128)
        t1 = jnp.dot(ea, w1, preferred_element_type=_F32).reshape(_T, _S4, 128) + b1
        g = _sp(t1).reshape(_T * _S4, 128)
        wf = jnp.dot(g, w2, preferred_element_type=_F32).reshape(_T, _S4, 128) + b2
        cw = 0.5 * (jnp.cos((ew * _PI) / _CUT) + 1.0)
        fac = jnp.where(m, cw, 0.0)
        xls = xlp_ref[pl.ds(base, _S4), :].reshape(1, _S4, 128)
        msg = (wf * fac) * xls
        return acc + jnp.sum(msg, axis=1)

    acc = jax.lax.fori_loop(0, cnt, step, jnp.zeros((_T, 128), _F32))
    aggr = jnp.dot(acc, q_ref[...], preferred_element_type=_F32)
    xc = jnp.dot(aggr, l2w_ref[...], preferred_element_type=_F32) + l2b_ref[0:1, :]
    xc = _sp(xc)
    xc = jnp.dot(xc, lw_ref[...], preferred_element_type=_F32) + lb_ref[0:1, :]
    o_ref[...] = h_ref[...] + xc


def _head_body(h_ref, o1w_ref, o1b_ref, o2w_ref, o2b_ref, o_ref):
    t = pl.program_id(0)

    @pl.when(t == 0)
    def _():
        o_ref[...] = jnp.zeros_like(o_ref)

    h = h_ref[...]
    u = _sp(jnp.dot(h, o1w_ref[...], preferred_element_type=_F32) + o1b_ref[0:1, :])
    y = jnp.dot(u, o2w_ref[...], preferred_element_type=_F32) + o2b_ref[0:1, :]
    localrow = jax.lax.broadcasted_iota(jnp.int32, (_T, 1), 0)
    y = jnp.where(localrow < _REAL, y, 0.0)
    o_ref[0:1, :] += jnp.sum(y, axis=0, keepdims=True)

    @pl.when(t == _NT - 1)
    def _():
        o_ref[...] = jnp.maximum(o_ref[...], 0.0)


def _pad8(b):
    return jnp.zeros((8, b.shape[0]), _F32).at[0].set(b.astype(_F32))


def _full(shape):
    return pl.BlockSpec(shape, lambda i, *_: tuple(0 for _ in shape))


def _rows(shape):
    return pl.BlockSpec(shape, lambda i, *_: (i,) + tuple(0 for _ in shape[1:]))


def kernel(z, pos, emb, mlp_w1, mlp_b1, mlp_w2, mlp_b2, lin1_w, lin2_w, lin2_b,
           lin_w, lin_b, out1_w, out1_b, out2_w, out2_b):
    pos = pos.astype(_F32)

    o1 = jnp.argsort(pos[:, 0]).reshape(5, 2000)
    o2 = jnp.argsort(pos[o1, 1], axis=1)
    p2 = jnp.take_along_axis(o1, o2, axis=1).reshape(20, 500)
    o3 = jnp.argsort(pos[p2, 2], axis=1)
    p3 = jnp.take_along_axis(p2, o3, axis=1).reshape(_NT, _REAL)

    pos_t = pos[p3]
    pos_t = jnp.concatenate(
        [pos_t, jnp.full((_NT, _T - _REAL, 3), 1e6, _F32)], axis=1)
    posf = pos_t.reshape(_NPAD, 3)
    z_t = z[p3].astype(jnp.int32)
    z_t = jnp.concatenate(
        [z_t, jnp.zeros((_NT, _T - _REAL), jnp.int32)], axis=1)
    zf = z_t.reshape(_NPAD, 1).astype(_F32)
    posr = jnp.pad(posf, ((0, 0), (0, 5)))
    xsp = jnp.repeat(posf[:, 0].reshape(-1, 4), 32, axis=1)
    ysp = jnp.repeat(posf[:, 1].reshape(-1, 4), 32, axis=1)
    zsp = jnp.repeat(posf[:, 2].reshape(-1, 4), 32, axis=1)

    vid = jnp.arange(_T).reshape(1, _T, 1) < _REAL
    pr = posf.reshape(_NT, _T, 3)
    lo = jnp.min(jnp.where(vid, pr, jnp.inf), axis=1)
    hi = jnp.max(jnp.where(vid, pr, -jnp.inf), axis=1)
    gap = jnp.maximum(0.0, jnp.maximum(lo[:, None, :] - hi[None, :, :],
                                       lo[None, :, :] - hi[:, None, :]))
    act = jnp.sum(gap * gap, axis=-1) <= (_CUT + 1e-2) ** 2
    cnt = jnp.sum(act, axis=1).astype(jnp.int32)
    lst = jnp.argsort(~act, axis=1, stable=True).astype(jnp.int32)

    embp = jnp.pad(emb.astype(_F32), ((0, 128 - emb.shape[0]), (0, 0)))
    offsp = _pad8(jnp.tile(jnp.asarray(_OFFS), 4))
    eye4 = jnp.eye(4, dtype=_F32)
    qmat = jnp.tile(jnp.eye(_NF, dtype=_F32), (4, 1))

    h = pl.pallas_call(
        _emb_body,
        grid=(_NT,),
        in_specs=[_rows((_T, 1)), _full((128, _HID))],
        out_specs=_rows((_T, _HID)),
        out_shape=jax.ShapeDtypeStruct((_NPAD, _HID), _F32),
    )(zf, embp)

    for i in range(_NI):
        xl = pl.pallas_call(
            _xl_body,
            grid=(_NT,),
            in_specs=[_rows((_T, _HID)), _full((_HID, _NF))],
            out_specs=_rows((_T, _NF)),
            out_shape=jax.ShapeDtypeStruct((_NPAD, _NF), _F32),
        )(h, lin1_w[i].astype(_F32))
        xlp = xl.reshape(_NPAD // 4, 128)

        w1p = jnp.pad(mlp_w1[i].astype(_F32), ((0, _NGP - _NG), (0, 0)))
        w1bd = jnp.kron(eye4, w1p)
        w2bd = jnp.kron(eye4, mlp_w2[i].astype(_F32))
        b1p = _pad8(jnp.tile(mlp_b1[i].astype(_F32), 4))
        b2p = _pad8(jnp.tile(mlp_b2[i].astype(_F32), 4))
        grid_spec = pltpu.PrefetchScalarGridSpec(
            num_scalar_prefetch=2,
            grid=(_NT,),
            in_specs=[
                _rows((_T, 8)),
                _full((_NPAD // 4, 128)),
                _full((_NPAD // 4, 128)),
                _full((_NPAD // 4, 128)),
                _full((_NPAD // 4, 128)),
                _rows((_T, _HID)),
                _full((8, 128)),
                _full((128, 128)),
                _full((8, 128)),
                _full((128, 128)),
                _full((8, 128)),
                _full((128, _NF)),
                _full((_NF, _HID)),
                _full((8, _HID)),
                _full((_HID, _HID)),
                _full((8, _HID)),
            ],
            out_specs=_rows((_T, _HID)),
        )
        h = pl.pallas_call(
            _msg_body,
            grid_spec=grid_spec,
            out_shape=jax.ShapeDtypeStruct((_NPAD, _HID), _F32),
        )(cnt, lst, posr, xsp, ysp, zsp, xlp, h,
          offsp, w1bd, b1p, w2bd, b2p, qmat,
          lin2_w[i].astype(_F32), _pad8(lin2_b[i]),
          lin_w[i].astype(_F32), _pad8(lin_b[i]))

    o8 = pl.pallas_call(
        _head_body,
        grid=(_NT,),
        in_specs=[_rows((_T, _HID)), _full((_HID, _HID // 2)),
                  _full((8, _HID // 2)), _full((_HID // 2, _NF)),
                  _full((8, _NF))],
        out_specs=_full((8, _NF)),
        out_shape=jax.ShapeDtypeStruct((8, _NF), _F32),
    )(h, out1_w.astype(_F32), _pad8(out1_b), out2_w.astype(_F32), _pad8(out2_b))
    return o8[0:1]


# speedup vs baseline: 2.9922x; 2.0549x over previous
import numpy as np
import jax
import jax.numpy as jnp
from jax.experimental import pallas as pl
from jax.experimental.pallas import tpu as pltpu

_N = 10000
_HID = 128
_NF = 32
_CUT = 2.5
_NG = 20
_NGP = 32
_NI = 3
_BOX = 27.3

_T = 128
_S = 128
_S4 = _S // 4
_NPAD = 10240
_NT = _NPAD // _T
_NC = _NPAD // _S
_REAL = 125

_off_np = np.linspace(0.0, _CUT, _NG).astype(np.float32)
_OFFS = np.full((_NGP,), 1e3, np.float32)
_OFFS[:_NG] = _off_np
_COEFF = np.float32(-0.5 / (_off_np[1] - _off_np[0]) ** 2)
_LOG2 = np.float32(np.log(2.0))
_PI = np.float32(np.pi)
_F32 = jnp.float32

_u = np.linspace(0.0, _CUT * _CUT + 0.02, 20001)
_cc = np.polynomial.chebyshev.chebfit(_u, np.cos((np.pi / _CUT) * np.sqrt(_u)), 8)
_CPOLY = np.polynomial.chebyshev.cheb2poly(_cc).astype(np.float32)


def _sp(x):
    return jnp.logaddexp(x, 0.0) - _LOG2


def _emb_body(zf_ref, emb_ref, o_ref):
    zf = zf_ref[...]
    cls = jax.lax.broadcasted_iota(jnp.int32, (_T, 128), 1).astype(_F32)
    oh = (zf == cls).astype(_F32)
    o_ref[...] = jnp.dot(oh, emb_ref[...], preferred_element_type=_F32)


def _xl_body(h_ref, w_ref, o_ref):
    o_ref[...] = jnp.dot(h_ref[...], w_ref[...], preferred_element_type=_F32)


def _prep_body(xsp_ref, ysp_ref, zsp_ref, o_ref):
    xs = xsp_ref[...]
    ys = ysp_ref[...]
    zs = zsp_ref[...]
    o_ref[...] = xs * xs + ys * ys + zs * zs


def _msg_body(cnt_ref, lst_ref, posr_ref, xsp_ref, ysp_ref, zsp_ref,
              sqsp_ref, colp_ref, xlp_ref,
              h_ref, offs_ref, w1_ref, b1_ref, w2_ref, b2_ref, q_ref,
              l2w_ref, l2b_ref, lw_ref, lb_ref, o_ref):
    t = pl.program_id(0)
    pd = posr_ref[...]
    xd = pd[:, 0:1].reshape(_T, 1, 1)
    yd = pd[:, 1:2].reshape(_T, 1, 1)
    zd = pd[:, 2:3].reshape(_T, 1, 1)
    sqd = xd * xd + yd * yd + zd * zd
    rowid = jax.lax.broadcasted_iota(jnp.int32, (_T, 1, 1), 0) + t * _T
    w1 = w1_ref[...]
    b1 = b1_ref[0:1, :].reshape(1, 1, 128)
    w2 = w2_ref[...]
    b2 = b2_ref[0:1, :].reshape(1, 1, 128)
    offs = offs_ref[0:1, :].reshape(1, 1, 128)
    cnt = cnt_ref[t]

    def step(s, acc):
        c = lst_ref[t, s]
        base = c * _S4
        xs = xsp_ref[pl.ds(base, _S4), :].reshape(1, _S4, 128)
        ys = ysp_ref[pl.ds(base, _S4), :].reshape(1, _S4, 128)
        zs = zsp_ref[pl.ds(base, _S4), :].reshape(1, _S4, 128)
        sqs = sqsp_ref[pl.ds(base, _S4), :].reshape(1, _S4, 128)
        colid = colp_ref[pl.ds(base, _S4), :].reshape(1, _S4, 128)
        dt = xd * xs + yd * ys + zd * zs
        d2 = sqd + sqs - 2.0 * dt
        m = (d2 < _CUT * _CUT) & (rowid != colid)
        dx = xs - xd
        dy = ys - yd
        dz = zs - zd
        u = dx * dx + dy * dy + dz * dz + 1e-12
        ew = jnp.sqrt(u)
        dlt = ew - offs
        ea = jnp.exp(_COEFF * (dlt * dlt)).reshape(_T * _S4, 128)
        t1 = jnp.dot(ea, w1, preferred_element_type=_F32).reshape(_T, _S4, 128) + b1
        g = _sp(t1).reshape(_T * _S4, 128)
        wf = jnp.dot(g, w2, preferred_element_type=_F32).reshape(_T, _S4, 128) + b2
        cw = jnp.full_like(u, float(_CPOLY[8]))
        for _c in _CPOLY[7::-1]:
            cw = cw * u + float(_c)
        cw = 0.5 * (cw + 1.0)
        fac = jnp.where(m, cw, 0.0)
        xls = xlp_ref[pl.ds(base, _S4), :].reshape(1, _S4, 128)
        msg = (wf * fac) * xls
        return acc + jnp.sum(msg, axis=1)

    acc = jax.lax.fori_loop(0, cnt, step, jnp.zeros((_T, 128), _F32))
    aggr = jnp.dot(acc, q_ref[...], preferred_element_type=_F32)
    xc = jnp.dot(aggr, l2w_ref[...], preferred_element_type=_F32) + l2b_ref[0:1, :]
    xc = _sp(xc)
    xc = jnp.dot(xc, lw_ref[...], preferred_element_type=_F32) + lb_ref[0:1, :]
    o_ref[...] = h_ref[...] + xc


def _head_body(h_ref, o1w_ref, o1b_ref, o2w_ref, o2b_ref, o_ref):
    t = pl.program_id(0)

    @pl.when(t == 0)
    def _():
        o_ref[...] = jnp.zeros_like(o_ref)

    h = h_ref[...]
    u = _sp(jnp.dot(h, o1w_ref[...], preferred_element_type=_F32) + o1b_ref[0:1, :])
    y = jnp.dot(u, o2w_ref[...], preferred_element_type=_F32) + o2b_ref[0:1, :]
    localrow = jax.lax.broadcasted_iota(jnp.int32, (_T, 1), 0)
    y = jnp.where(localrow < _REAL, y, 0.0)
    o_ref[0:1, :] += jnp.sum(y, axis=0, keepdims=True)

    @pl.when(t == _NT - 1)
    def _():
        o_ref[...] = jnp.maximum(o_ref[...], 0.0)


def _pad8(b):
    return jnp.zeros((8, b.shape[0]), _F32).at[0].set(b.astype(_F32))


def _full(shape):
    return pl.BlockSpec(shape, lambda i, *_: tuple(0 for _ in shape))


def _rows(shape):
    return pl.BlockSpec(shape, lambda i, *_: (i,) + tuple(0 for _ in shape[1:]))


def kernel(z, pos, emb, mlp_w1, mlp_b1, mlp_w2, mlp_b2, lin1_w, lin2_w, lin2_b,
           lin_w, lin_b, out1_w, out1_b, out2_w, out2_b):
    pos = pos.astype(_F32)

    o1 = jnp.argsort(pos[:, 0]).reshape(5, 2000)
    o2 = jnp.argsort(pos[o1, 1], axis=1)
    p2 = jnp.take_along_axis(o1, o2, axis=1).reshape(20, 500)
    o3 = jnp.argsort(pos[p2, 2], axis=1)
    p3 = jnp.take_along_axis(p2, o3, axis=1).reshape(_NT, _REAL)

    pos_t = pos[p3]
    pos_t = jnp.concatenate(
        [pos_t, jnp.full((_NT, _T - _REAL, 3), 1e6, _F32)], axis=1)
    posf = pos_t.reshape(_NPAD, 3)
    z_t = z[p3].astype(jnp.int32)
    z_t = jnp.concatenate(
        [z_t, jnp.zeros((_NT, _T - _REAL), jnp.int32)], axis=1)
    zf = z_t.reshape(_NPAD, 1).astype(_F32)
    posr = jnp.pad(posf, ((0, 0), (0, 5)))
    xsp = jnp.repeat(posf[:, 0].reshape(-1, 4), 32, axis=1)
    ysp = jnp.repeat(posf[:, 1].reshape(-1, 4), 32, axis=1)
    zsp = jnp.repeat(posf[:, 2].reshape(-1, 4), 32, axis=1)
    colp = jnp.repeat(jnp.arange(_NPAD, dtype=jnp.int32).reshape(-1, 4),
                      32, axis=1)

    vid = jnp.arange(_T).reshape(1, _T, 1) < _REAL
    pr = posf.reshape(_NT, _T, 3)
    lo = jnp.min(jnp.where(vid, pr, jnp.inf), axis=1)
    hi = jnp.max(jnp.where(vid, pr, -jnp.inf), axis=1)
    gap = jnp.maximum(0.0, jnp.maximum(lo[:, None, :] - hi[None, :, :],
                                       lo[None, :, :] - hi[:, None, :]))
    act = jnp.sum(gap * gap, axis=-1) <= (_CUT + 1e-2) ** 2
    cnt = jnp.sum(act, axis=1).astype(jnp.int32)
    lst = jnp.argsort(~act, axis=1, stable=True).astype(jnp.int32)

    embp = jnp.pad(emb.astype(_F32), ((0, 128 - emb.shape[0]), (0, 0)))
    offsp = _pad8(jnp.tile(jnp.asarray(_OFFS), 4))
    eye4 = jnp.eye(4, dtype=_F32)
    qmat = jnp.tile(jnp.eye(_NF, dtype=_F32), (4, 1))

    sqsp = pl.pallas_call(
        _prep_body,
        grid=(1,),
        in_specs=[_full((_NPAD // 4, 128))] * 3,
        out_specs=_full((_NPAD // 4, 128)),
        out_shape=jax.ShapeDtypeStruct((_NPAD // 4, 128), _F32),
    )(xsp, ysp, zsp)

    h = pl.pallas_call(
        _emb_body,
        grid=(_NT,),
        in_specs=[_rows((_T, 1)), _full((128, _HID))],
        out_specs=_rows((_T, _HID)),
        out_shape=jax.ShapeDtypeStruct((_NPAD, _HID), _F32),
    )(zf, embp)

    for i in range(_NI):
        xl = pl.pallas_call(
            _xl_body,
            grid=(_NT,),
            in_specs=[_rows((_T, _HID)), _full((_HID, _NF))],
            out_specs=_rows((_T, _NF)),
            out_shape=jax.ShapeDtypeStruct((_NPAD, _NF), _F32),
        )(h, lin1_w[i].astype(_F32))
        xlp = xl.reshape(_NPAD // 4, 128)

        w1p = jnp.pad(mlp_w1[i].astype(_F32), ((0, _NGP - _NG), (0, 0)))
        w1bd = jnp.kron(eye4, w1p)
        w2bd = jnp.kron(eye4, mlp_w2[i].astype(_F32))
        b1p = _pad8(jnp.tile(mlp_b1[i].astype(_F32), 4))
        b2p = _pad8(jnp.tile(mlp_b2[i].astype(_F32), 4))
        grid_spec = pltpu.PrefetchScalarGridSpec(
            num_scalar_prefetch=2,
            grid=(_NT,),
            in_specs=[
                _rows((_T, 8)),
                _full((_NPAD // 4, 128)),
                _full((_NPAD // 4, 128)),
                _full((_NPAD // 4, 128)),
                _full((_NPAD // 4, 128)),
                _full((_NPAD // 4, 128)),
                _full((_NPAD // 4, 128)),
                _rows((_T, _HID)),
                _full((8, 128)),
                _full((128, 128)),
                _full((8, 128)),
                _full((128, 128)),
                _full((8, 128)),
                _full((128, _NF)),
                _full((_NF, _HID)),
                _full((8, _HID)),
                _full((_HID, _HID)),
                _full((8, _HID)),
            ],
            out_specs=_rows((_T, _HID)),
        )
        h = pl.pallas_call(
            _msg_body,
            grid_spec=grid_spec,
            out_shape=jax.ShapeDtypeStruct((_NPAD, _HID), _F32),
        )(cnt, lst, posr, xsp, ysp, zsp, sqsp, colp, xlp, h,
          offsp, w1bd, b1p, w2bd, b2p, qmat,
          lin2_w[i].astype(_F32), _pad8(lin2_b[i]),
          lin_w[i].astype(_F32), _pad8(lin_b[i]))

    o8 = pl.pallas_call(
        _head_body,
        grid=(_NT,),
        in_specs=[_rows((_T, _HID)), _full((_HID, _HID // 2)),
                  _full((8, _HID // 2)), _full((_HID // 2, _NF)),
                  _full((8, _NF))],
        out_specs=_full((8, _NF)),
        out_shape=jax.ShapeDtypeStruct((8, _NF), _F32),
    )(h, out1_w.astype(_F32), _pad8(out1_b), out2_w.astype(_F32), _pad8(out2_b))
    return o8[0:1]


# speedup vs baseline: 3.2568x; 1.0884x over previous
import numpy as np
import jax
import jax.numpy as jnp
from jax.experimental import pallas as pl
from jax.experimental.pallas import tpu as pltpu

_N = 10000
_HID = 128
_NF = 32
_CUT = 2.5
_NG = 20
_NGP = 32
_NI = 3
_BOX = 27.3

_T = 128
_S = 128
_S4 = _S // 4
_NPAD = 10240
_NT = _NPAD // _T
_NC = _NPAD // _S
_REAL = 125

_off_np = np.linspace(0.0, _CUT, _NG).astype(np.float32)
_OFFS = np.full((_NGP,), 1e3, np.float32)
_OFFS[:_NG] = _off_np
_COEFF = np.float32(-0.5 / (_off_np[1] - _off_np[0]) ** 2)
_LOG2 = np.float32(np.log(2.0))
_PI = np.float32(np.pi)
_F32 = jnp.float32

_u = np.linspace(0.0, _CUT * _CUT + 0.02, 20001)
_cc = np.polynomial.chebyshev.chebfit(_u, np.cos((np.pi / _CUT) * np.sqrt(_u)), 8)
_CPOLY = np.polynomial.chebyshev.cheb2poly(_cc).astype(np.float32)


def _sp(x):
    return jnp.logaddexp(x, 0.0) - _LOG2


def _emb_body(zf_ref, emb_ref, o_ref):
    zf = zf_ref[...]
    cls = jax.lax.broadcasted_iota(jnp.int32, (_T, 128), 1).astype(_F32)
    oh = (zf == cls).astype(_F32)
    o_ref[...] = jnp.dot(oh, emb_ref[...], preferred_element_type=_F32)


def _xl_body(h_ref, w_ref, o_ref):
    o_ref[...] = jnp.dot(h_ref[...], w_ref[...], preferred_element_type=_F32)


def _prep_body(xsp_ref, ysp_ref, zsp_ref, o_ref):
    xs = xsp_ref[...]
    ys = ysp_ref[...]
    zs = zsp_ref[...]
    o_ref[...] = xs * xs + ys * ys + zs * zs


def _msg_body(cnt_ref, lst_ref, posr_ref, xsp_ref, ysp_ref, zsp_ref,
              sqsp_ref, colp_ref, xlp_ref,
              h_ref, offs_ref, w1_ref, b1_ref, w2_ref, b2_ref, q_ref,
              l2w_ref, l2b_ref, lw_ref, lb_ref, o_ref):
    t = pl.program_id(0)
    pd = posr_ref[...]
    xd = pd[:, 0:1].reshape(_T, 1, 1)
    yd = pd[:, 1:2].reshape(_T, 1, 1)
    zd = pd[:, 2:3].reshape(_T, 1, 1)
    sqd = xd * xd + yd * yd + zd * zd
    rowid = jax.lax.broadcasted_iota(jnp.int32, (_T, 1, 1), 0) + t * _T
    w1 = w1_ref[...]
    b1 = b1_ref[0:1, :].reshape(1, 1, 128)
    w2 = w2_ref[...]
    b2 = b2_ref[0:1, :].reshape(1, 1, 128)
    offs = offs_ref[0:1, :].reshape(1, 1, 128)
    cnt = cnt_ref[t]

    def step(s, acc):
        c = lst_ref[t, s]
        base = c * _S4
        xs = xsp_ref[pl.ds(base, _S4), :].reshape(1, _S4, 128)
        ys = ysp_ref[pl.ds(base, _S4), :].reshape(1, _S4, 128)
        zs = zsp_ref[pl.ds(base, _S4), :].reshape(1, _S4, 128)
        sqs = sqsp_ref[pl.ds(base, _S4), :].reshape(1, _S4, 128)
        colid = colp_ref[pl.ds(base, _S4), :].reshape(1, _S4, 128)
        dt = xd * xs + yd * ys + zd * zs
        d2 = sqd + sqs - 2.0 * dt
        m = (d2 < _CUT * _CUT) & (rowid != colid)
        u = jnp.maximum(d2, 0.0) + 1e-12
        ew = jnp.sqrt(u)
        dlt = ew - offs
        ea = jnp.exp(_COEFF * (dlt * dlt)).reshape(_T * _S4, 128)
        t1 = jnp.dot(ea, w1, preferred_element_type=_F32).reshape(_T, _S4, 128) + b1
        g = _sp(t1).reshape(_T * _S4, 128)
        wf = jnp.dot(g, w2, preferred_element_type=_F32).reshape(_T, _S4, 128) + b2
        cw = jnp.full_like(u, float(_CPOLY[8]))
        for _c in _CPOLY[7::-1]:
            cw = cw * u + float(_c)
        cw = 0.5 * (cw + 1.0)
        fac = jnp.where(m, cw, 0.0)
        xls = xlp_ref[pl.ds(base, _S4), :].reshape(1, _S4, 128)
        msg = (wf * fac) * xls
        return acc + jnp.sum(msg, axis=1)

    acc = jax.lax.fori_loop(0, cnt, step, jnp.zeros((_T, 128), _F32))
    aggr = jnp.dot(acc, q_ref[...], preferred_element_type=_F32)
    xc = jnp.dot(aggr, l2w_ref[...], preferred_element_type=_F32) + l2b_ref[0:1, :]
    xc = _sp(xc)
    xc = jnp.dot(xc, lw_ref[...], preferred_element_type=_F32) + lb_ref[0:1, :]
    o_ref[...] = h_ref[...] + xc


def _head_body(h_ref, o1w_ref, o1b_ref, o2w_ref, o2b_ref, o_ref):
    t = pl.program_id(0)

    @pl.when(t == 0)
    def _():
        o_ref[...] = jnp.zeros_like(o_ref)

    h = h_ref[...]
    u = _sp(jnp.dot(h, o1w_ref[...], preferred_element_type=_F32) + o1b_ref[0:1, :])
    y = jnp.dot(u, o2w_ref[...], preferred_element_type=_F32) + o2b_ref[0:1, :]
    localrow = jax.lax.broadcasted_iota(jnp.int32, (_T, 1), 0)
    y = jnp.where(localrow < _REAL, y, 0.0)
    o_ref[0:1, :] += jnp.sum(y, axis=0, keepdims=True)

    @pl.when(t == _NT - 1)
    def _():
        o_ref[...] = jnp.maximum(o_ref[...], 0.0)


def _pad8(b):
    return jnp.zeros((8, b.shape[0]), _F32).at[0].set(b.astype(_F32))


def _full(shape):
    return pl.BlockSpec(shape, lambda i, *_: tuple(0 for _ in shape))


def _rows(shape):
    return pl.BlockSpec(shape, lambda i, *_: (i,) + tuple(0 for _ in shape[1:]))


def kernel(z, pos, emb, mlp_w1, mlp_b1, mlp_w2, mlp_b2, lin1_w, lin2_w, lin2_b,
           lin_w, lin_b, out1_w, out1_b, out2_w, out2_b):
    pos = pos.astype(_F32)

    o1 = jnp.argsort(pos[:, 0]).reshape(5, 2000)
    o2 = jnp.argsort(pos[o1, 1], axis=1)
    p2 = jnp.take_along_axis(o1, o2, axis=1).reshape(20, 500)
    o3 = jnp.argsort(pos[p2, 2], axis=1)
    p3 = jnp.take_along_axis(p2, o3, axis=1).reshape(_NT, _REAL)

    pos_t = pos[p3]
    pos_t = jnp.concatenate(
        [pos_t, jnp.full((_NT, _T - _REAL, 3), 1e6, _F32)], axis=1)
    posf = pos_t.reshape(_NPAD, 3)
    z_t = z[p3].astype(jnp.int32)
    z_t = jnp.concatenate(
        [z_t, jnp.zeros((_NT, _T - _REAL), jnp.int32)], axis=1)
    zf = z_t.reshape(_NPAD, 1).astype(_F32)
    posr = jnp.pad(posf, ((0, 0), (0, 5)))
    xsp = jnp.repeat(posf[:, 0].reshape(-1, 4), 32, axis=1)
    ysp = jnp.repeat(posf[:, 1].reshape(-1, 4), 32, axis=1)
    zsp = jnp.repeat(posf[:, 2].reshape(-1, 4), 32, axis=1)
    colp = jnp.repeat(jnp.arange(_NPAD, dtype=jnp.int32).reshape(-1, 4),
                      32, axis=1)

    vid = jnp.arange(_T).reshape(1, _T, 1) < _REAL
    pr = posf.reshape(_NT, _T, 3)
    lo = jnp.min(jnp.where(vid, pr, jnp.inf), axis=1)
    hi = jnp.max(jnp.where(vid, pr, -jnp.inf), axis=1)
    gap = jnp.maximum(0.0, jnp.maximum(lo[:, None, :] - hi[None, :, :],
                                       lo[None, :, :] - hi[:, None, :]))
    act = jnp.sum(gap * gap, axis=-1) <= (_CUT + 1e-2) ** 2
    cnt = jnp.sum(act, axis=1).astype(jnp.int32)
    lst = jnp.argsort(~act, axis=1, stable=True).astype(jnp.int32)

    embp = jnp.pad(emb.astype(_F32), ((0, 128 - emb.shape[0]), (0, 0)))
    offsp = _pad8(jnp.tile(jnp.asarray(_OFFS), 4))
    eye4 = jnp.eye(4, dtype=_F32)
    qmat = jnp.tile(jnp.eye(_NF, dtype=_F32), (4, 1))

    sqsp = pl.pallas_call(
        _prep_body,
        grid=(1,),
        in_specs=[_full((_NPAD // 4, 128))] * 3,
        out_specs=_full((_NPAD // 4, 128)),
        out_shape=jax.ShapeDtypeStruct((_NPAD // 4, 128), _F32),
    )(xsp, ysp, zsp)

    h = pl.pallas_call(
        _emb_body,
        grid=(_NT,),
        in_specs=[_rows((_T, 1)), _full((128, _HID))],
        out_specs=_rows((_T, _HID)),
        out_shape=jax.ShapeDtypeStruct((_NPAD, _HID), _F32),
    )(zf, embp)

    for i in range(_NI):
        xl = pl.pallas_call(
            _xl_body,
            grid=(_NT,),
            in_specs=[_rows((_T, _HID)), _full((_HID, _NF))],
            out_specs=_rows((_T, _NF)),
            out_shape=jax.ShapeDtypeStruct((_NPAD, _NF), _F32),
        )(h, lin1_w[i].astype(_F32))
        xlp = xl.reshape(_NPAD // 4, 128)

        w1p = jnp.pad(mlp_w1[i].astype(_F32), ((0, _NGP - _NG), (0, 0)))
        w1bd = jnp.kron(eye4, w1p)
        w2bd = jnp.kron(eye4, mlp_w2[i].astype(_F32))
        b1p = _pad8(jnp.tile(mlp_b1[i].astype(_F32), 4))
        b2p = _pad8(jnp.tile(mlp_b2[i].astype(_F32), 4))
        grid_spec = pltpu.PrefetchScalarGridSpec(
            num_scalar_prefetch=2,
            grid=(_NT,),
            in_specs=[
                _rows((_T, 8)),
                _full((_NPAD // 4, 128)),
                _full((_NPAD // 4, 128)),
                _full((_NPAD // 4, 128)),
                _full((_NPAD // 4, 128)),
                _full((_NPAD // 4, 128)),
                _full((_NPAD // 4, 128)),
                _rows((_T, _HID)),
                _full((8, 128)),
                _full((128, 128)),
                _full((8, 128)),
                _full((128, 128)),
                _full((8, 128)),
                _full((128, _NF)),
                _full((_NF, _HID)),
                _full((8, _HID)),
                _full((_HID, _HID)),
                _full((8, _HID)),
            ],
            out_specs=_rows((_T, _HID)),
        )
        h = pl.pallas_call(
            _msg_body,
            grid_spec=grid_spec,
            out_shape=jax.ShapeDtypeStruct((_NPAD, _HID), _F32),
        )(cnt, lst, posr, xsp, ysp, zsp, sqsp, colp, xlp, h,
          offsp, w1bd, b1p, w2bd, b2p, qmat,
          lin2_w[i].astype(_F32), _pad8(lin2_b[i]),
          lin_w[i].astype(_F32), _pad8(lin_b[i]))

    o8 = pl.pallas_call(
        _head_body,
        grid=(_NT,),
        in_specs=[_rows((_T, _HID)), _full((_HID, _HID // 2)),
                  _full((8, _HID // 2)), _full((_HID // 2, _NF)),
                  _full((8, _NF))],
        out_specs=_full((8, _NF)),
        out_shape=jax.ShapeDtypeStruct((8, _NF), _F32),
    )(h, out1_w.astype(_F32), _pad8(out1_b), out2_w.astype(_F32), _pad8(out2_b))
    return o8[0:1]


# speedup vs baseline: 5.3673x; 1.6480x over previous
import numpy as np
import jax
import jax.numpy as jnp
from jax.experimental import pallas as pl
from jax.experimental.pallas import tpu as pltpu

_N = 10000
_HID = 128
_NF = 32
_CUT = 2.5
_NG = 20
_NGP = 32
_NI = 3
_BOX = 27.3

_T = 64
_S = 32
_S4 = _S // 4
_NPAD = 10240
_NT = _NPAD // _T
_NC = _NPAD // _S
_TN = 128
_NTN = _NPAD // _TN

_off_np = np.linspace(0.0, _CUT, _NG).astype(np.float32)
_OFFS = np.full((_NGP,), 1e3, np.float32)
_OFFS[:_NG] = _off_np
_COEFF = np.float32(-0.5 / (_off_np[1] - _off_np[0]) ** 2)
_LOG2 = np.float32(np.log(2.0))
_PI = np.float32(np.pi)
_F32 = jnp.float32

_u = np.linspace(0.0, _CUT * _CUT + 0.02, 20001)
_cc = np.polynomial.chebyshev.chebfit(_u, np.cos((np.pi / _CUT) * np.sqrt(_u)), 8)
_CPOLY = np.polynomial.chebyshev.cheb2poly(_cc).astype(np.float32)


def _sp(x):
    return jnp.logaddexp(x, 0.0) - _LOG2


def _emb_body(zf_ref, emb_ref, o_ref):
    zf = zf_ref[...]
    cls = jax.lax.broadcasted_iota(jnp.int32, (_TN, 128), 1).astype(_F32)
    oh = (zf == cls).astype(_F32)
    o_ref[...] = jnp.dot(oh, emb_ref[...], preferred_element_type=_F32)


def _xl_body(h_ref, w_ref, o_ref):
    o_ref[...] = jnp.dot(h_ref[...], w_ref[...], preferred_element_type=_F32)


def _prep_body(xsp_ref, ysp_ref, zsp_ref, o_ref):
    xs = xsp_ref[...]
    ys = ysp_ref[...]
    zs = zsp_ref[...]
    o_ref[...] = xs * xs + ys * ys + zs * zs


def _msg_body(cnt_ref, lst_ref, posr_ref, xsp_ref, ysp_ref, zsp_ref,
              sqsp_ref, colp_ref, xlp_ref,
              h_ref, offs_ref, w1_ref, b1_ref, w2_ref, b2_ref, q_ref,
              l2w_ref, l2b_ref, lw_ref, lb_ref, o_ref):
    t = pl.program_id(0)
    pd = posr_ref[...]
    xd = pd[:, 0:1].reshape(_T, 1, 1)
    yd = pd[:, 1:2].reshape(_T, 1, 1)
    zd = pd[:, 2:3].reshape(_T, 1, 1)
    sqd = xd * xd + yd * yd + zd * zd
    rowid = jax.lax.broadcasted_iota(jnp.int32, (_T, 1, 1), 0) + t * _T
    w1 = w1_ref[...]
    b1 = b1_ref[0:1, :].reshape(1, 1, 128)
    w2 = w2_ref[...]
    b2 = b2_ref[0:1, :].reshape(1, 1, 128)
    offs = offs_ref[0:1, :].reshape(1, 1, 128)
    cnt = cnt_ref[t]

    def step(s, acc):
        c = lst_ref[t, s]
        base = c * _S4
        xs = xsp_ref[pl.ds(base, _S4), :].reshape(1, _S4, 128)
        ys = ysp_ref[pl.ds(base, _S4), :].reshape(1, _S4, 128)
        zs = zsp_ref[pl.ds(base, _S4), :].reshape(1, _S4, 128)
        sqs = sqsp_ref[pl.ds(base, _S4), :].reshape(1, _S4, 128)
        colid = colp_ref[pl.ds(base, _S4), :].reshape(1, _S4, 128)
        dt = xd * xs + yd * ys + zd * zs
        d2 = sqd + sqs - 2.0 * dt
        m = (d2 < _CUT * _CUT) & (rowid != colid)
        u = jnp.maximum(d2, 0.0) + 1e-12
        ew = jnp.sqrt(u)
        dlt = ew - offs
        ea = jnp.exp(_COEFF * (dlt * dlt)).reshape(_T * _S4, 128)
        t1 = jnp.dot(ea, w1, preferred_element_type=_F32).reshape(_T, _S4, 128) + b1
        g = _sp(t1).reshape(_T * _S4, 128)
        wf = jnp.dot(g, w2, preferred_element_type=_F32).reshape(_T, _S4, 128) + b2
        cw = jnp.full_like(u, float(_CPOLY[8]))
        for _c in _CPOLY[7::-1]:
            cw = cw * u + float(_c)
        cw = 0.5 * (cw + 1.0)
        fac = jnp.where(m, cw, 0.0)
        xls = xlp_ref[pl.ds(base, _S4), :].reshape(1, _S4, 128)
        msg = (wf * fac) * xls
        return acc + jnp.sum(msg, axis=1)

    acc = jax.lax.fori_loop(0, cnt, step, jnp.zeros((_T, 128), _F32))
    aggr = jnp.dot(acc, q_ref[...], preferred_element_type=_F32)
    xc = jnp.dot(aggr, l2w_ref[...], preferred_element_type=_F32) + l2b_ref[0:1, :]
    xc = _sp(xc)
    xc = jnp.dot(xc, lw_ref[...], preferred_element_type=_F32) + lb_ref[0:1, :]
    o_ref[...] = h_ref[...] + xc


def _head_body(h_ref, o1w_ref, o1b_ref, o2w_ref, o2b_ref, o_ref):
    t = pl.program_id(0)

    @pl.when(t == 0)
    def _():
        o_ref[...] = jnp.zeros_like(o_ref)

    h = h_ref[...]
    u = _sp(jnp.dot(h, o1w_ref[...], preferred_element_type=_F32) + o1b_ref[0:1, :])
    y = jnp.dot(u, o2w_ref[...], preferred_element_type=_F32) + o2b_ref[0:1, :]
    rowid = jax.lax.broadcasted_iota(jnp.int32, (_TN, 1), 0) + t * _TN
    y = jnp.where(rowid < _N, y, 0.0)
    o_ref[0:1, :] += jnp.sum(y, axis=0, keepdims=True)

    @pl.when(t == _NTN - 1)
    def _():
        o_ref[...] = jnp.maximum(o_ref[...], 0.0)


def _pad8(b):
    return jnp.zeros((8, b.shape[0]), _F32).at[0].set(b.astype(_F32))


def _full(shape):
    return pl.BlockSpec(shape, lambda i, *_: tuple(0 for _ in shape))


def _rows(shape):
    return pl.BlockSpec(shape, lambda i, *_: (i,) + tuple(0 for _ in shape[1:]))


def kernel(z, pos, emb, mlp_w1, mlp_b1, mlp_w2, mlp_b2, lin1_w, lin2_w, lin2_b,
           lin_w, lin_b, out1_w, out1_b, out2_w, out2_b):
    pos = pos.astype(_F32)

    pos0 = jnp.concatenate([pos, jnp.full((_NPAD - _N, 3), 1e6, _F32)], axis=0)
    z0 = jnp.concatenate([z.astype(jnp.int32),
                          jnp.zeros((_NPAD - _N,), jnp.int32)])
    o1 = jnp.argsort(pos0[:, 0]).reshape(8, 1280)
    o2 = jnp.argsort(pos0[o1, 1], axis=1)
    p2 = jnp.take_along_axis(o1, o2, axis=1).reshape(40, 256)
    o3 = jnp.argsort(pos0[p2, 2], axis=1)
    p3 = jnp.take_along_axis(p2, o3, axis=1).reshape(_NPAD)

    posf = pos0[p3]
    zf = z0[p3].reshape(_NPAD, 1).astype(_F32)
    posr = jnp.pad(posf, ((0, 0), (0, 5)))
    xsp = jnp.repeat(posf[:, 0].reshape(-1, 4), 32, axis=1)
    ysp = jnp.repeat(posf[:, 1].reshape(-1, 4), 32, axis=1)
    zsp = jnp.repeat(posf[:, 2].reshape(-1, 4), 32, axis=1)
    colp = jnp.repeat(jnp.arange(_NPAD, dtype=jnp.int32).reshape(-1, 4),
                      32, axis=1)

    vid = (jnp.arange(_NPAD) < _N).reshape(_NC, _S, 1)
    pr = posf.reshape(_NC, _S, 3)
    slo = jnp.min(jnp.where(vid, pr, jnp.inf), axis=1)
    shi = jnp.max(jnp.where(vid, pr, -jnp.inf), axis=1)
    dlo = jnp.min(slo.reshape(_NT, _T // _S, 3), axis=1)
    dhi = jnp.max(shi.reshape(_NT, _T // _S, 3), axis=1)
    gap = jnp.maximum(0.0, jnp.maximum(dlo[:, None, :] - shi[None, :, :],
                                       slo[None, :, :] - dhi[:, None, :]))
    act = jnp.sum(gap * gap, axis=-1) <= (_CUT + 1e-2) ** 2
    cnt = jnp.sum(act, axis=1).astype(jnp.int32)
    lst = jnp.argsort(~act, axis=1, stable=True).astype(jnp.int32)

    embp = jnp.pad(emb.astype(_F32), ((0, 128 - emb.shape[0]), (0, 0)))
    offsp = _pad8(jnp.tile(jnp.asarray(_OFFS), 4))
    eye4 = jnp.eye(4, dtype=_F32)
    qmat = jnp.tile(jnp.eye(_NF, dtype=_F32), (4, 1))

    sqsp = pl.pallas_call(
        _prep_body,
        grid=(1,),
        in_specs=[_full((_NPAD // 4, 128))] * 3,
        out_specs=_full((_NPAD // 4, 128)),
        out_shape=jax.ShapeDtypeStruct((_NPAD // 4, 128), _F32),
    )(xsp, ysp, zsp)

    h = pl.pallas_call(
        _emb_body,
        grid=(_NTN,),
        in_specs=[_rows((_TN, 1)), _full((128, _HID))],
        out_specs=_rows((_TN, _HID)),
        out_shape=jax.ShapeDtypeStruct((_NPAD, _HID), _F32),
    )(zf, embp)

    for i in range(_NI):
        xl = pl.pallas_call(
            _xl_body,
            grid=(_NTN,),
            in_specs=[_rows((_TN, _HID)), _full((_HID, _NF))],
            out_specs=_rows((_TN, _NF)),
            out_shape=jax.ShapeDtypeStruct((_NPAD, _NF), _F32),
        )(h, lin1_w[i].astype(_F32))
        xlp = xl.reshape(_NPAD // 4, 128)

        w1p = jnp.pad(mlp_w1[i].astype(_F32), ((0, _NGP - _NG), (0, 0)))
        w1bd = jnp.kron(eye4, w1p)
        w2bd = jnp.kron(eye4, mlp_w2[i].astype(_F32))
        b1p = _pad8(jnp.tile(mlp_b1[i].astype(_F32), 4))
        b2p = _pad8(jnp.tile(mlp_b2[i].astype(_F32), 4))
        grid_spec = pltpu.PrefetchScalarGridSpec(
            num_scalar_prefetch=2,
            grid=(_NT,),
            in_specs=[
                _rows((_T, 8)),
                _full((_NPAD // 4, 128)),
                _full((_NPAD // 4, 128)),
                _full((_NPAD // 4, 128)),
                _full((_NPAD // 4, 128)),
                _full((_NPAD // 4, 128)),
                _full((_NPAD // 4, 128)),
                _rows((_T, _HID)),
                _full((8, 128)),
                _full((128, 128)),
                _full((8, 128)),
                _full((128, 128)),
                _full((8, 128)),
                _full((128, _NF)),
                _full((_NF, _HID)),
                _full((8, _HID)),
                _full((_HID, _HID)),
                _full((8, _HID)),
            ],
            out_specs=_rows((_T, _HID)),
        )
        h = pl.pallas_call(
            _msg_body,
            grid_spec=grid_spec,
            out_shape=jax.ShapeDtypeStruct((_NPAD, _HID), _F32),
        )(cnt, lst, posr, xsp, ysp, zsp, sqsp, colp, xlp, h,
          offsp, w1bd, b1p, w2bd, b2p, qmat,
          lin2_w[i].astype(_F32), _pad8(lin2_b[i]),
          lin_w[i].astype(_F32), _pad8(lin_b[i]))

    o8 = pl.pallas_call(
        _head_body,
        grid=(_NTN,),
        in_specs=[_rows((_TN, _HID)), _full((_HID, _HID // 2)),
                  _full((8, _HID // 2)), _full((_HID // 2, _NF)),
                  _full((8, _NF))],
        out_specs=_full((8, _NF)),
        out_shape=jax.ShapeDtypeStruct((8, _NF), _F32),
    )(h, out1_w.astype(_F32), _pad8(out1_b), out2_w.astype(_F32), _pad8(out2_b))
    return o8[0:1]


# speedup vs baseline: 5.3831x; 1.0029x over previous
import numpy as np
import jax
import jax.numpy as jnp
from jax.experimental import pallas as pl
from jax.experimental.pallas import tpu as pltpu
import jax.experimental.pallas.tpu_sc as plsc

_N = 10000
_HID = 128
_NF = 32
_CUT = 2.5
_NG = 20
_NGP = 32
_NI = 3
_BOX = 27.3

_T = 64
_S = 32
_S4 = _S // 4
_NPAD = 10240
_NT = _NPAD // _T
_NC = _NPAD // _S
_TN = 128
_NTN = _NPAD // _TN

_off_np = np.linspace(0.0, _CUT, _NG).astype(np.float32)
_OFFS = np.full((_NGP,), 1e3, np.float32)
_OFFS[:_NG] = _off_np
_COEFF = np.float32(-0.5 / (_off_np[1] - _off_np[0]) ** 2)
_LOG2 = np.float32(np.log(2.0))
_PI = np.float32(np.pi)
_F32 = jnp.float32

_u = np.linspace(0.0, _CUT * _CUT + 0.02, 20001)
_cc = np.polynomial.chebyshev.chebfit(_u, np.cos((np.pi / _CUT) * np.sqrt(_u)), 8)
_CPOLY = np.polynomial.chebyshev.cheb2poly(_cc).astype(np.float32)


def _sp(x):
    return jnp.logaddexp(x, 0.0) - _LOG2


_VMESH = plsc.VectorSubcoreMesh(core_axis_name="core", subcore_axis_name="subcore")


def _emb_gather(embp, zi):
    @pl.kernel(out_type=jax.ShapeDtypeStruct((_NPAD, _HID), _F32),
               mesh=_VMESH)
    def _k(x_hbm, i_hbm, o_hbm):
        def body(i_vmem, o_vmem):
            pltpu.sync_copy(x_hbm.at[i_vmem.at[0]], o_vmem)

        pltpu.emit_pipeline(
            body,
            grid=(_NPAD // 128,),
            in_specs=[pl.BlockSpec((1, 128), index_map=lambda i: (0, i))],
            out_specs=[pl.BlockSpec((128, _HID), index_map=lambda i: (i, 0))],
            core_axis_name="subcore",
            dimension_semantics=(pltpu.PARALLEL,),
        )(i_hbm, o_hbm)

    return _k(embp, zi)


def _xl_body(h_ref, w_ref, o_ref):
    o_ref[...] = jnp.dot(h_ref[...], w_ref[...], preferred_element_type=_F32)


def _prep_body(xsp_ref, ysp_ref, zsp_ref, o_ref):
    xs = xsp_ref[...]
    ys = ysp_ref[...]
    zs = zsp_ref[...]
    o_ref[...] = xs * xs + ys * ys + zs * zs


def _msg_body(cnt_ref, lst_ref, posr_ref, xsp_ref, ysp_ref, zsp_ref,
              sqsp_ref, colp_ref, xlp_ref,
              h_ref, offs_ref, w1_ref, b1_ref, w2_ref, b2_ref, q_ref,
              l2w_ref, l2b_ref, lw_ref, lb_ref, o_ref):
    t = pl.program_id(0)
    pd = posr_ref[...]
    xd = pd[:, 0:1].reshape(_T, 1, 1)
    yd = pd[:, 1:2].reshape(_T, 1, 1)
    zd = pd[:, 2:3].reshape(_T, 1, 1)
    sqd = xd * xd + yd * yd + zd * zd
    rowid = jax.lax.broadcasted_iota(jnp.int32, (_T, 1, 1), 0) + t * _T
    w1 = w1_ref[...]
    b1 = b1_ref[0:1, :].reshape(1, 1, 128)
    w2 = w2_ref[...]
    b2 = b2_ref[0:1, :].reshape(1, 1, 128)
    offs = offs_ref[0:1, :].reshape(1, 1, 128)
    cnt = cnt_ref[t]

    def step(s, acc):
        c = lst_ref[t, s]
        base = c * _S4
        xs = xsp_ref[pl.ds(base, _S4), :].reshape(1, _S4, 128)
        ys = ysp_ref[pl.ds(base, _S4), :].reshape(1, _S4, 128)
        zs = zsp_ref[pl.ds(base, _S4), :].reshape(1, _S4, 128)
        sqs = sqsp_ref[pl.ds(base, _S4), :].reshape(1, _S4, 128)
        colid = colp_ref[pl.ds(base, _S4), :].reshape(1, _S4, 128)
        dt = xd * xs + yd * ys + zd * zs
        d2 = sqd + sqs - 2.0 * dt
        m = (d2 < _CUT * _CUT) & (rowid != colid)
        u = jnp.maximum(d2, 0.0) + 1e-12
        ew = jnp.sqrt(u)
        dlt = ew - offs
        ea = jnp.exp(_COEFF * (dlt * dlt)).reshape(_T * _S4, 128)
        t1 = jnp.dot(ea, w1, preferred_element_type=_F32).reshape(_T, _S4, 128) + b1
        g = _sp(t1).reshape(_T * _S4, 128)
        wf = jnp.dot(g, w2, preferred_element_type=_F32).reshape(_T, _S4, 128) + b2
        cw = jnp.full_like(u, float(_CPOLY[8]))
        for _c in _CPOLY[7::-1]:
            cw = cw * u + float(_c)
        cw = 0.5 * (cw + 1.0)
        fac = jnp.where(m, cw, 0.0)
        xls = xlp_ref[pl.ds(base, _S4), :].reshape(1, _S4, 128)
        msg = (wf * fac) * xls
        return acc + jnp.sum(msg, axis=1)

    acc = jax.lax.fori_loop(0, cnt, step, jnp.zeros((_T, 128), _F32))
    aggr = jnp.dot(acc, q_ref[...], preferred_element_type=_F32)
    xc = jnp.dot(aggr, l2w_ref[...], preferred_element_type=_F32) + l2b_ref[0:1, :]
    xc = _sp(xc)
    xc = jnp.dot(xc, lw_ref[...], preferred_element_type=_F32) + lb_ref[0:1, :]
    o_ref[...] = h_ref[...] + xc


def _head_body(h_ref, o1w_ref, o1b_ref, o2w_ref, o2b_ref, o_ref):
    t = pl.program_id(0)

    @pl.when(t == 0)
    def _():
        o_ref[...] = jnp.zeros_like(o_ref)

    h = h_ref[...]
    u = _sp(jnp.dot(h, o1w_ref[...], preferred_element_type=_F32) + o1b_ref[0:1, :])
    y = jnp.dot(u, o2w_ref[...], preferred_element_type=_F32) + o2b_ref[0:1, :]
    rowid = jax.lax.broadcasted_iota(jnp.int32, (_TN, 1), 0) + t * _TN
    y = jnp.where(rowid < _N, y, 0.0)
    o_ref[0:1, :] += jnp.sum(y, axis=0, keepdims=True)

    @pl.when(t == _NTN - 1)
    def _():
        o_ref[...] = jnp.maximum(o_ref[...], 0.0)


def _pad8(b):
    return jnp.zeros((8, b.shape[0]), _F32).at[0].set(b.astype(_F32))


def _full(shape):
    return pl.BlockSpec(shape, lambda i, *_: tuple(0 for _ in shape))


def _rows(shape):
    return pl.BlockSpec(shape, lambda i, *_: (i,) + tuple(0 for _ in shape[1:]))


def kernel(z, pos, emb, mlp_w1, mlp_b1, mlp_w2, mlp_b2, lin1_w, lin2_w, lin2_b,
           lin_w, lin_b, out1_w, out1_b, out2_w, out2_b):
    pos = pos.astype(_F32)

    pos0 = jnp.concatenate([pos, jnp.full((_NPAD - _N, 3), 1e6, _F32)], axis=0)
    z0 = jnp.concatenate([z.astype(jnp.int32),
                          jnp.zeros((_NPAD - _N,), jnp.int32)])
    o1 = jnp.argsort(pos0[:, 0]).reshape(8, 1280)
    o2 = jnp.argsort(pos0[o1, 1], axis=1)
    p2 = jnp.take_along_axis(o1, o2, axis=1).reshape(40, 256)
    o3 = jnp.argsort(pos0[p2, 2], axis=1)
    p3 = jnp.take_along_axis(p2, o3, axis=1).reshape(_NPAD)

    posf = pos0[p3]
    zi = z0[p3].reshape(1, _NPAD)
    posr = jnp.pad(posf, ((0, 0), (0, 5)))
    xsp = jnp.repeat(posf[:, 0].reshape(-1, 4), 32, axis=1)
    ysp = jnp.repeat(posf[:, 1].reshape(-1, 4), 32, axis=1)
    zsp = jnp.repeat(posf[:, 2].reshape(-1, 4), 32, axis=1)
    colp = jnp.repeat(jnp.arange(_NPAD, dtype=jnp.int32).reshape(-1, 4),
                      32, axis=1)

    vid = (jnp.arange(_NPAD) < _N).reshape(_NC, _S, 1)
    pr = posf.reshape(_NC, _S, 3)
    slo = jnp.min(jnp.where(vid, pr, jnp.inf), axis=1)
    shi = jnp.max(jnp.where(vid, pr, -jnp.inf), axis=1)
    dlo = jnp.min(slo.reshape(_NT, _T // _S, 3), axis=1)
    dhi = jnp.max(shi.reshape(_NT, _T // _S, 3), axis=1)
    gap = jnp.maximum(0.0, jnp.maximum(dlo[:, None, :] - shi[None, :, :],
                                       slo[None, :, :] - dhi[:, None, :]))
    act = jnp.sum(gap * gap, axis=-1) <= (_CUT + 1e-2) ** 2
    cnt = jnp.sum(act, axis=1).astype(jnp.int32)
    lst = jnp.argsort(~act, axis=1, stable=True).astype(jnp.int32)

    embp = jnp.pad(emb.astype(_F32), ((0, 128 - emb.shape[0]), (0, 0)))
    offsp = _pad8(jnp.tile(jnp.asarray(_OFFS), 4))
    eye4 = jnp.eye(4, dtype=_F32)
    qmat = jnp.tile(jnp.eye(_NF, dtype=_F32), (4, 1))

    sqsp = pl.pallas_call(
        _prep_body,
        grid=(1,),
        in_specs=[_full((_NPAD // 4, 128))] * 3,
        out_specs=_full((_NPAD // 4, 128)),
        out_shape=jax.ShapeDtypeStruct((_NPAD // 4, 128), _F32),
    )(xsp, ysp, zsp)

    h = _emb_gather(embp, zi)

    for i in range(_NI):
        xl = pl.pallas_call(
            _xl_body,
            grid=(_NTN,),
            in_specs=[_rows((_TN, _HID)), _full((_HID, _NF))],
            out_specs=_rows((_TN, _NF)),
            out_shape=jax.ShapeDtypeStruct((_NPAD, _NF), _F32),
        )(h, lin1_w[i].astype(_F32))
        xlp = xl.reshape(_NPAD // 4, 128)

        w1p = jnp.pad(mlp_w1[i].astype(_F32), ((0, _NGP - _NG), (0, 0)))
        w1bd = jnp.kron(eye4, w1p)
        w2bd = jnp.kron(eye4, mlp_w2[i].astype(_F32))
        b1p = _pad8(jnp.tile(mlp_b1[i].astype(_F32), 4))
        b2p = _pad8(jnp.tile(mlp_b2[i].astype(_F32), 4))
        grid_spec = pltpu.PrefetchScalarGridSpec(
            num_scalar_prefetch=2,
            grid=(_NT,),
            in_specs=[
                _rows((_T, 8)),
                _full((_NPAD // 4, 128)),
                _full((_NPAD // 4, 128)),
                _full((_NPAD // 4, 128)),
                _full((_NPAD // 4, 128)),
                _full((_NPAD // 4, 128)),
                _full((_NPAD // 4, 128)),
                _rows((_T, _HID)),
                _full((8, 128)),
                _full((128, 128)),
                _full((8, 128)),
                _full((128, 128)),
                _full((8, 128)),
                _full((128, _NF)),
                _full((_NF, _HID)),
                _full((8, _HID)),
                _full((_HID, _HID)),
                _full((8, _HID)),
            ],
            out_specs=_rows((_T, _HID)),
        )
        h = pl.pallas_call(
            _msg_body,
            grid_spec=grid_spec,
            out_shape=jax.ShapeDtypeStruct((_NPAD, _HID), _F32),
        )(cnt, lst, posr, xsp, ysp, zsp, sqsp, colp, xlp, h,
          offsp, w1bd, b1p, w2bd, b2p, qmat,
          lin2_w[i].astype(_F32), _pad8(lin2_b[i]),
          lin_w[i].astype(_F32), _pad8(lin_b[i]))

    o8 = pl.pallas_call(
        _head_body,
        grid=(_NTN,),
        in_specs=[_rows((_TN, _HID)), _full((_HID, _HID // 2)),
                  _full((8, _HID // 2)), _full((_HID // 2, _NF)),
                  _full((8, _NF))],
        out_specs=_full((8, _NF)),
        out_shape=jax.ShapeDtypeStruct((8, _NF), _F32),
    )(h, out1_w.astype(_F32), _pad8(out1_b), out2_w.astype(_F32), _pad8(out2_b))
    return o8[0:1]
